# Initial kernel scaffold; baseline (speedup 1.0000x reference)
import jax
import jax.numpy as jnp
from jax.experimental import pallas as pl


def kernel(x, edge_index, aff_w, cog_w, W1, b1, W2, b2):
    raise NotImplementedError("write your pallas kernel here")


# speedup vs baseline: 23.4244x; 23.4244x over previous
import functools

import jax
import jax.numpy as jnp
from jax import lax
from jax.experimental import pallas as pl
from jax.experimental.pallas import tpu as pltpu
from jax.experimental.pallas import tpu_sc as plsc

N = 100000
E = 6400000
IN_DIM = 5
HIDDEN = 128
NUM_CLASSES = 2

NC = 2
NS = 16
NW = NC * NS

D = 16
NP = 100352
K = 128
EPT = 200064
EPAD = EPT * NW
NB = EPT // K
ROWS_PER_TILE = NP // NS

BK = 2048

_mesh = plsc.VectorSubcoreMesh(
    core_axis_name="c", subcore_axis_name="s", num_cores=NC, num_subcores=NS)


def _hist_body(dst_hbm, out_hbm, counts, dbuf):
  c = lax.axis_index("c")
  s = lax.axis_index("s")
  wid = s * NC + c

  zeros16 = jnp.zeros((16,), jnp.float32)
  def zero_body(i, carry):
    counts[pl.ds(i * 16, 16)] = zeros16
    return carry
  lax.fori_loop(0, NP // 16, zero_body, 0)

  ones16 = jnp.ones((16,), jnp.float32)
  base = wid * EPT
  def blk(j, carry):
    pltpu.sync_copy(dst_hbm.at[pl.ds(base + j * K, K)], dbuf)
    for t in range(K // 16):
      idx = dbuf[pl.ds(t * 16, 16)]
      plsc.addupdate_scatter(counts, [idx], ones16)
    return carry
  lax.fori_loop(0, NB, blk, 0)

  pltpu.sync_copy(counts, out_hbm.at[wid])


_hist_kernel = functools.partial(
    pl.kernel,
    out_type=jax.ShapeDtypeStruct((NW, NP), jnp.float32),
    mesh=_mesh,
    scratch_types=[
        pltpu.VMEM((NP,), jnp.float32),
        pltpu.VMEM((K,), jnp.int32),
    ],
    compiler_params=pltpu.CompilerParams(needs_layout_passes=False),
)(_hist_body)


def _prop_body(src_hbm, dst_hbm, g_hbm, zrows_hbm, out_hbm,
               acc, sbuf, dbuf, rows, sem):
  c = lax.axis_index("c")
  s = lax.axis_index("s")
  wid = s * NC + c
  rs = s * ROWS_PER_TILE

  pltpu.sync_copy(zrows_hbm, acc.at[pl.ds(rs, ROWS_PER_TILE)])
  plsc.subcore_barrier()

  base = wid * EPT
  def blk(j, carry):
    b = base + j * K
    pltpu.sync_copy(src_hbm.at[pl.ds(b, K)], sbuf.at[0])
    pltpu.sync_copy(dst_hbm.at[pl.ds(b, K)], dbuf.at[0])
    pltpu.async_copy(g_hbm.at[sbuf.at[0]], rows, sem).wait()
    pltpu.sync_copy(rows, acc.at[dbuf.at[0]], add=True)
    return carry
  lax.fori_loop(0, NB, blk, 0)

  plsc.subcore_barrier()
  pltpu.sync_copy(acc.at[pl.ds(rs, ROWS_PER_TILE)],
                  out_hbm.at[c, pl.ds(rs, ROWS_PER_TILE)])


_prop_kernel = functools.partial(
    pl.kernel,
    out_type=jax.ShapeDtypeStruct((NC, NP, D), jnp.float32),
    mesh=_mesh,
    scratch_types=[
        pltpu.VMEM_SHARED((NP, D), jnp.float32),
        pltpu.VMEM((1, K), jnp.int32),
        pltpu.VMEM((1, K), jnp.int32),
        pltpu.VMEM((K, D), jnp.float32),
        pltpu.SemaphoreType.DMA,
    ],
    compiler_params=pltpu.CompilerParams(
        needs_layout_passes=False, use_tc_tiling_on_sc=False),
)(_prop_body)


def _deg_body(hist_ref, x_ref, g_ref, dis_ref):
  deg = 1.0 + jnp.sum(hist_ref[...], axis=0)
  dis = lax.rsqrt(deg)
  dis_ref[...] = dis[:, None]
  g_ref[...] = x_ref[...] * dis[:, None]


def _deg_scale(hist, xpad):
  return pl.pallas_call(
      _deg_body,
      grid=(NP // BK,),
      in_specs=[
          pl.BlockSpec((NW, BK), lambda i: (0, i)),
          pl.BlockSpec((BK, D), lambda i: (i, 0)),
      ],
      out_specs=[
          pl.BlockSpec((BK, D), lambda i: (i, 0)),
          pl.BlockSpec((BK, 1), lambda i: (i, 0)),
      ],
      out_shape=[
          jax.ShapeDtypeStruct((NP, D), jnp.float32),
          jax.ShapeDtypeStruct((NP, 1), jnp.float32),
      ],
  )(hist, xpad)


def _mlp_body(acc_ref, g1_ref, dis_ref, w1_ref, b1_ref, w2_ref, g2_ref):
  dis = dis_ref[...]
  p = (acc_ref[0] + acc_ref[1] + g1_ref[...]) * dis
  h1 = jnp.maximum(
      jnp.dot(p, w1_ref[...], preferred_element_type=jnp.float32)
      + b1_ref[...], 0.0)
  t = jnp.dot(h1, w2_ref[...], preferred_element_type=jnp.float32)
  g2_ref[...] = t * dis


def _mlp(acc1, g1, dis, W1p, b1, W2p):
  return pl.pallas_call(
      _mlp_body,
      grid=(NP // BK,),
      in_specs=[
          pl.BlockSpec((NC, BK, D), lambda i: (0, i, 0)),
          pl.BlockSpec((BK, D), lambda i: (i, 0)),
          pl.BlockSpec((BK, 1), lambda i: (i, 0)),
          pl.BlockSpec((D, HIDDEN), lambda i: (0, 0)),
          pl.BlockSpec((1, HIDDEN), lambda i: (0, 0)),
          pl.BlockSpec((HIDDEN, D), lambda i: (0, 0)),
      ],
      out_specs=pl.BlockSpec((BK, D), lambda i: (i, 0)),
      out_shape=jax.ShapeDtypeStruct((NP, D), jnp.float32),
  )(acc1, g1, dis, W1p, b1.reshape(1, HIDDEN), W2p)


def _out_body(acc_ref, g2_ref, dis_ref, b2_ref, out_ref):
  q = (acc_ref[0] + acc_ref[1] + g2_ref[...]) * dis_ref[...]
  q = q[:, :NUM_CLASSES] + b2_ref[...]
  m = jnp.max(q, axis=1, keepdims=True)
  lse = m + jnp.log(jnp.sum(jnp.exp(q - m), axis=1, keepdims=True))
  out_ref[...] = q - lse


def _logits(acc2, g2, dis, b2):
  return pl.pallas_call(
      _out_body,
      grid=(NP // BK,),
      in_specs=[
          pl.BlockSpec((NC, BK, D), lambda i: (0, i, 0)),
          pl.BlockSpec((BK, D), lambda i: (i, 0)),
          pl.BlockSpec((BK, 1), lambda i: (i, 0)),
          pl.BlockSpec((1, NUM_CLASSES), lambda i: (0, 0)),
      ],
      out_specs=pl.BlockSpec((BK, NUM_CLASSES), lambda i: (i, 0)),
      out_shape=jax.ShapeDtypeStruct((NP, NUM_CLASSES), jnp.float32),
  )(acc2, g2, dis, b2.reshape(1, NUM_CLASSES))


@jax.jit
def kernel(x, edge_index, aff_w, cog_w, W1, b1, W2, b2):
  del aff_w, cog_w

  pad = jnp.full((EPAD - E,), N, dtype=jnp.int32)
  srcp = jnp.concatenate([edge_index[0], pad])
  dstp = jnp.concatenate([edge_index[1], pad])

  xpad = jnp.zeros((NP, D), jnp.float32).at[:N, :IN_DIM].set(x)
  W1p = jnp.zeros((D, HIDDEN), jnp.float32).at[:IN_DIM].set(W1)
  W2p = jnp.zeros((HIDDEN, D), jnp.float32).at[:, :NUM_CLASSES].set(W2)
  zrows = jnp.zeros((ROWS_PER_TILE, D), jnp.float32)

  hist = _hist_kernel(dstp)
  g1, dis = _deg_scale(hist, xpad)
  acc1 = _prop_kernel(srcp, dstp, g1, zrows)
  g2 = _mlp(acc1, g1, dis, W1p, b1, W2p)
  acc2 = _prop_kernel(srcp, dstp, g2, zrows)
  out = _logits(acc2, g2, dis, b2)
  return out[:N]


# speedup vs baseline: 98.1353x; 4.1895x over previous
import functools

import jax
import jax.numpy as jnp
from jax import lax
from jax.experimental import pallas as pl
from jax.experimental.pallas import tpu as pltpu
from jax.experimental.pallas import tpu_sc as plsc

N = 100000
E = 6400000
IN_DIM = 5
HIDDEN = 128
NUM_CLASSES = 2

NC = 2
NS = 16
NW = NC * NS

D = 8
NP = 100352
K = 128
SB = 8
EPT = 200704
EPAD = EPT * NW
NSB = EPT // (K * SB)
RPT = EPT // K
NCH = EPT // 2048
ROWS_PER_TILE = NP // NS
NSLOT = 4

BK = 2048

_mesh = plsc.VectorSubcoreMesh(
    core_axis_name="c", subcore_axis_name="s", num_cores=NC, num_subcores=NS)


def _hist_body(dst_hbm, out_hbm, counts, dbuf, sem):
  c = lax.axis_index("c")
  s = lax.axis_index("s")
  wid = s * NC + c

  zeros16 = jnp.zeros((16,), jnp.float32)
  def zero_body(i, carry):
    counts[pl.ds(i * 16, 16)] = zeros16
    return carry
  lax.fori_loop(0, NP // 16, zero_body, 0)

  ones16 = jnp.ones((16,), jnp.float32)
  base = wid * EPT

  def fire(u, slot):
    pltpu.make_async_copy(
        dst_hbm.at[pl.ds(base + u * 2048, 2048)], dbuf.at[slot], sem).start()

  def drain(slot):
    pltpu.make_async_copy(
        dst_hbm.at[pl.ds(base, 2048)], dbuf.at[slot], sem).wait()

  fire(0, 0)
  def chunk(u, carry):
    slot = lax.rem(u, 2)
    drain(slot)
    @pl.when(u + 1 < NCH)
    def _():
      fire(u + 1, 1 - slot)
    for t in range(2048 // 16):
      idx = dbuf[slot, pl.ds(t * 16, 16)]
      plsc.addupdate_scatter(counts, [idx], ones16)
    return carry
  lax.fori_loop(0, NCH, chunk, 0)

  pltpu.sync_copy(counts, out_hbm.at[wid])


_hist_kernel = functools.partial(
    pl.kernel,
    out_type=jax.ShapeDtypeStruct((NW, NP), jnp.float32),
    mesh=_mesh,
    scratch_types=[
        pltpu.VMEM((NP,), jnp.float32),
        pltpu.VMEM((2, 2048), jnp.int32),
        pltpu.SemaphoreType.DMA,
    ],
    compiler_params=pltpu.CompilerParams(needs_layout_passes=False),
)(_hist_body)


def _prop_body(src_hbm, dst_hbm, g_hbm, zrows_hbm, out_hbm,
               acc, sbuf, dbuf, rows, sem_i, sem_g, sem_s):
  c = lax.axis_index("c")
  s = lax.axis_index("s")
  wid = s * NC + c
  rs = s * ROWS_PER_TILE

  pltpu.sync_copy(zrows_hbm, acc.at[pl.ds(rs, ROWS_PER_TILE)])
  plsc.subcore_barrier()

  base_row = wid * RPT

  def fire_idx(j, slot):
    r0 = base_row + j * SB
    pltpu.make_async_copy(
        src_hbm.at[pl.ds(r0, SB)], sbuf.at[slot], sem_i).start()
    pltpu.make_async_copy(
        dst_hbm.at[pl.ds(r0, SB)], dbuf.at[slot], sem_i).start()

  def wait_idx(slot):
    pltpu.make_async_copy(
        src_hbm.at[pl.ds(base_row, SB)], sbuf.at[slot], sem_i).wait()
    pltpu.make_async_copy(
        dst_hbm.at[pl.ds(base_row, SB)], dbuf.at[slot], sem_i).wait()

  def gather(slot, b):
    return pltpu.make_async_copy(
        g_hbm.at[sbuf.at[slot, b]], rows.at[slot, b], sem_g)

  def scatter(slot, b):
    return pltpu.make_async_copy(
        rows.at[slot, b], acc.at[dbuf.at[slot, b]], sem_s)

  fire_idx(0, 0)
  fire_idx(1, 1)

  def superblock(j, carry):
    slot = lax.rem(j, NSLOT)
    wait_idx(slot)
    for b in range(SB):
      gather(slot, b).start()
    for b in range(SB):
      gather(slot, b).wait()
      scatter(slot, b).start(add=True)
    @pl.when(j >= 1)
    def _():
      pslot = lax.rem(j + NSLOT - 1, NSLOT)
      for b in range(SB):
        scatter(pslot, b).wait()
    @pl.when(j + 2 < NSB)
    def _():
      fire_idx(j + 2, lax.rem(j + 2, NSLOT))
    return carry
  lax.fori_loop(0, NSB, superblock, 0)

  for b in range(SB):
    scatter((NSB - 1) % NSLOT, b).wait()

  plsc.subcore_barrier()
  pltpu.sync_copy(acc.at[pl.ds(rs, ROWS_PER_TILE)],
                  out_hbm.at[c, pl.ds(rs, ROWS_PER_TILE)])


_prop_kernel = functools.partial(
    pl.kernel,
    out_type=jax.ShapeDtypeStruct((NC, NP, D), jnp.float32),
    mesh=_mesh,
    scratch_types=[
        pltpu.VMEM_SHARED((NP, D), jnp.float32),
        pltpu.VMEM((NSLOT, SB, K), jnp.int32),
        pltpu.VMEM((NSLOT, SB, K), jnp.int32),
        pltpu.VMEM((NSLOT, SB, K, D), jnp.float32),
        pltpu.SemaphoreType.DMA,
        pltpu.SemaphoreType.DMA,
        pltpu.SemaphoreType.DMA,
    ],
    compiler_params=pltpu.CompilerParams(
        needs_layout_passes=False, use_tc_tiling_on_sc=False),
)(_prop_body)


def _deg_body(hist_ref, x_ref, g_ref, dis_ref):
  deg = 1.0 + jnp.sum(hist_ref[...], axis=0)
  dis = lax.rsqrt(deg)
  dis_ref[...] = dis[:, None]
  g_ref[...] = x_ref[...] * dis[:, None]


def _deg_scale(hist, xpad):
  return pl.pallas_call(
      _deg_body,
      grid=(NP // BK,),
      in_specs=[
          pl.BlockSpec((NW, BK), lambda i: (0, i)),
          pl.BlockSpec((BK, D), lambda i: (i, 0)),
      ],
      out_specs=[
          pl.BlockSpec((BK, D), lambda i: (i, 0)),
          pl.BlockSpec((BK, 1), lambda i: (i, 0)),
      ],
      out_shape=[
          jax.ShapeDtypeStruct((NP, D), jnp.float32),
          jax.ShapeDtypeStruct((NP, 1), jnp.float32),
      ],
  )(hist, xpad)


def _mlp_body(acc_ref, g1_ref, dis_ref, w1_ref, b1_ref, w2_ref, g2_ref):
  dis = dis_ref[...]
  p = (acc_ref[0] + acc_ref[1] + g1_ref[...]) * dis
  h1 = jnp.maximum(
      jnp.dot(p, w1_ref[...], preferred_element_type=jnp.float32)
      + b1_ref[...], 0.0)
  t = jnp.dot(h1, w2_ref[...], preferred_element_type=jnp.float32)
  g2_ref[...] = t * dis


def _mlp(acc1, g1, dis, W1p, b1, W2p):
  return pl.pallas_call(
      _mlp_body,
      grid=(NP // BK,),
      in_specs=[
          pl.BlockSpec((NC, BK, D), lambda i: (0, i, 0)),
          pl.BlockSpec((BK, D), lambda i: (i, 0)),
          pl.BlockSpec((BK, 1), lambda i: (i, 0)),
          pl.BlockSpec((D, HIDDEN), lambda i: (0, 0)),
          pl.BlockSpec((1, HIDDEN), lambda i: (0, 0)),
          pl.BlockSpec((HIDDEN, D), lambda i: (0, 0)),
      ],
      out_specs=pl.BlockSpec((BK, D), lambda i: (i, 0)),
      out_shape=jax.ShapeDtypeStruct((NP, D), jnp.float32),
  )(acc1, g1, dis, W1p, b1.reshape(1, HIDDEN), W2p)


def _out_body(acc_ref, g2_ref, dis_ref, b2_ref, out_ref):
  q = (acc_ref[0] + acc_ref[1] + g2_ref[...]) * dis_ref[...]
  q = q[:, :NUM_CLASSES] + b2_ref[...]
  m = jnp.max(q, axis=1, keepdims=True)
  lse = m + jnp.log(jnp.sum(jnp.exp(q - m), axis=1, keepdims=True))
  out_ref[...] = q - lse


def _logits(acc2, g2, dis, b2):
  return pl.pallas_call(
      _out_body,
      grid=(NP // BK,),
      in_specs=[
          pl.BlockSpec((NC, BK, D), lambda i: (0, i, 0)),
          pl.BlockSpec((BK, D), lambda i: (i, 0)),
          pl.BlockSpec((BK, 1), lambda i: (i, 0)),
          pl.BlockSpec((1, NUM_CLASSES), lambda i: (0, 0)),
      ],
      out_specs=pl.BlockSpec((BK, NUM_CLASSES), lambda i: (i, 0)),
      out_shape=jax.ShapeDtypeStruct((NP, NUM_CLASSES), jnp.float32),
  )(acc2, g2, dis, b2.reshape(1, NUM_CLASSES))


@jax.jit
def kernel(x, edge_index, aff_w, cog_w, W1, b1, W2, b2):
  del aff_w, cog_w

  pad = jnp.full((EPAD - E,), N, dtype=jnp.int32)
  srcp = jnp.concatenate([edge_index[0], pad])
  dstp = jnp.concatenate([edge_index[1], pad])
  src2 = srcp.reshape(EPAD // K, K)
  dst2 = dstp.reshape(EPAD // K, K)

  xpad = jnp.zeros((NP, D), jnp.float32).at[:N, :IN_DIM].set(x)
  W1p = jnp.zeros((D, HIDDEN), jnp.float32).at[:IN_DIM].set(W1)
  W2p = jnp.zeros((HIDDEN, D), jnp.float32).at[:, :NUM_CLASSES].set(W2)
  zrows = jnp.zeros((ROWS_PER_TILE, D), jnp.float32)

  hist = _hist_kernel(dstp)
  g1, dis = _deg_scale(hist, xpad)
  acc1 = _prop_kernel(src2, dst2, g1, zrows)
  g2 = _mlp(acc1, g1, dis, W1p, b1, W2p)
  acc2 = _prop_kernel(src2, dst2, g2, zrows)
  out = _logits(acc2, g2, dis, b2)
  return out[:N]
